# baseline (speedup 1.0000x reference)
import functools

import jax
import jax.numpy as jnp
from jax import lax
from jax.experimental import pallas as pl
from jax.experimental.pallas import tpu as pltpu
from jax.experimental.pallas import tpu_sc as plsc

B, S, D = 4, 4096, 1024
NW = 32
WPB = NW // B
RPW = (B * S) // NW
R = 32
NCH = RPW // R
JN = D // 16


def _sc_add_body(x_hbm, ts_hbm, table_hbm, out_hbm,
                 tsv, emb4, buf0, buf1, buf2,
                 gsem, si0, si1, si2, so0, so1, so2):
    cid = lax.axis_index("c")
    sid = lax.axis_index("s")
    wid = cid * 16 + sid
    b = wid // WPB

    pltpu.sync_copy(ts_hbm, tsv)
    pltpu.async_copy(table_hbm.at[tsv], emb4, gsem).wait()

    row0 = wid * RPW
    bufs = (buf0, buf1, buf2)
    sins = (si0, si1, si2)
    souts = (so0, so1, so2)

    def start_in(c):
        s = c % 3
        return pltpu.async_copy(
            x_hbm.at[pl.ds(row0 + c * R, R)], bufs[s], sins[s])

    def start_out(c):
        s = c % 3
        return pltpu.async_copy(
            bufs[s], out_hbm.at[pl.ds(row0 + c * R, R)], souts[s])

    def compute(c):
        buf = bufs[c % 3]

        def row(r, carry):
            for j in range(JN):
                sl = pl.ds(j * 16, 16)
                buf[r, sl] = buf[r, sl] + emb4[b, sl]
            return carry

        lax.fori_loop(0, R, row, 0)

    hin = {0: start_in(0), 1: start_in(1)}
    hout = {}
    for c in range(NCH):
        hin[c].wait()
        compute(c)
        hout[c] = start_out(c)
        nxt = c + 2
        if nxt < NCH:
            if nxt - 3 >= 0:
                hout[nxt - 3].wait()
            hin[nxt] = start_in(nxt)
    for c in range(max(0, NCH - 3), NCH):
        hout[c].wait()


def _sc_add(x2, ts, table):
    mesh = plsc.VectorSubcoreMesh(core_axis_name="c", subcore_axis_name="s")
    f = functools.partial(
        pl.kernel, mesh=mesh,
        out_type=jax.ShapeDtypeStruct((B * S, D), jnp.float32),
        scratch_types=[
            pltpu.VMEM((4,), jnp.int32),
            pltpu.VMEM((4, D), jnp.float32),
            pltpu.VMEM((R, D), jnp.float32),
            pltpu.VMEM((R, D), jnp.float32),
            pltpu.VMEM((R, D), jnp.float32),
            pltpu.SemaphoreType.DMA,
            pltpu.SemaphoreType.DMA,
            pltpu.SemaphoreType.DMA,
            pltpu.SemaphoreType.DMA,
            pltpu.SemaphoreType.DMA,
            pltpu.SemaphoreType.DMA,
            pltpu.SemaphoreType.DMA,
        ],
    )(_sc_add_body)
    return f(x2, ts, table)


def kernel(x, timestep, embed_table):
    ts = timestep.astype(jnp.int32)
    x2 = x.reshape(B * S, D)
    out2 = _sc_add(x2, ts, embed_table)
    return out2.reshape(B, S, D)


# speedup vs baseline: 3.8716x; 3.8716x over previous
import functools

import jax
import jax.numpy as jnp
from jax import lax
from jax.experimental import pallas as pl
from jax.experimental.pallas import tpu as pltpu
from jax.experimental.pallas import tpu_sc as plsc

B, S, D = 4, 4096, 1024
NW = 32
WPB = NW // B
RPW = (B * S) // NW
R = 32
NCH = RPW // R
JN = D // 16


def _sc_add_body(x_hbm, ts_hbm, table_hbm, out_hbm,
                 tsv, emb4, buf0, buf1, buf2,
                 gsem, si0, si1, si2, so0, so1, so2):
    cid = lax.axis_index("c")
    sid = lax.axis_index("s")
    wid = cid * 16 + sid
    b = wid // WPB

    pltpu.sync_copy(ts_hbm, tsv)
    pltpu.async_copy(table_hbm.at[tsv], emb4, gsem).wait()

    row0 = wid * RPW
    bufs = (buf0, buf1, buf2)
    sins = (si0, si1, si2)
    souts = (so0, so1, so2)

    def start_in(c):
        s = c % 3
        return pltpu.async_copy(
            x_hbm.at[pl.ds(row0 + c * R, R)], bufs[s], sins[s])

    def start_out(c):
        s = c % 3
        return pltpu.async_copy(
            bufs[s], out_hbm.at[pl.ds(row0 + c * R, R)], souts[s])

    def compute(c):
        buf = bufs[c % 3]

        def row(r, carry):
            for j in range(JN):
                sl = pl.ds(j * 16, 16)
                buf[r, sl] = buf[r, sl] + emb4[b, sl]
            return carry

        lax.fori_loop(0, R, row, 0)

    hin = {0: start_in(0), 1: start_in(1)}
    hout = {}
    for c in range(NCH):
        hin[c].wait()
        compute(c)
        hout[c] = start_out(c)
        nxt = c + 2
        if nxt < NCH:
            if nxt - 3 >= 0:
                hout[nxt - 3].wait()
            hin[nxt] = start_in(nxt)
    for c in range(max(0, NCH - 3), NCH):
        hout[c].wait()


def _sc_add(x2, ts, table):
    mesh = plsc.VectorSubcoreMesh(core_axis_name="c", subcore_axis_name="s")
    f = functools.partial(
        pl.kernel, mesh=mesh,
        out_type=jax.ShapeDtypeStruct((B * S, D), jnp.float32),
        scratch_types=[
            pltpu.VMEM((4,), jnp.int32),
            pltpu.VMEM((4, D), jnp.float32),
            pltpu.VMEM((R, D), jnp.float32),
            pltpu.VMEM((R, D), jnp.float32),
            pltpu.VMEM((R, D), jnp.float32),
            pltpu.SemaphoreType.DMA,
            pltpu.SemaphoreType.DMA,
            pltpu.SemaphoreType.DMA,
            pltpu.SemaphoreType.DMA,
            pltpu.SemaphoreType.DMA,
            pltpu.SemaphoreType.DMA,
            pltpu.SemaphoreType.DMA,
        ],
    )(_sc_add_body)
    return f(x2, ts, table)


def _tc_add_body(ts_ref, x_ref, emb_ref, o_ref):
    o_ref[...] = x_ref[...] + emb_ref[...]


def _tc_add(x, ts, embed_table):
    BS = 2048
    table3 = embed_table.reshape(embed_table.shape[0], 1, D)
    return pl.pallas_call(
        _tc_add_body,
        grid_spec=pltpu.PrefetchScalarGridSpec(
            num_scalar_prefetch=1,
            grid=(B, S // BS),
            in_specs=[
                pl.BlockSpec((1, BS, D), lambda b, s, ts_ref: (b, s, 0)),
                pl.BlockSpec((1, 1, D), lambda b, s, ts_ref: (ts_ref[b], 0, 0)),
            ],
            out_specs=pl.BlockSpec((1, BS, D), lambda b, s, ts_ref: (b, s, 0)),
        ),
        out_shape=jax.ShapeDtypeStruct((B, S, D), x.dtype),
    )(ts, x, table3)


def kernel(x, timestep, embed_table):
    ts = timestep.astype(jnp.int32)
    return _tc_add(x, ts, embed_table)
